# Initial kernel scaffold; baseline (speedup 1.0000x reference)
import jax
import jax.numpy as jnp
from jax.experimental import pallas as pl


def kernel(x, graph_edge_index, edge_pairs, W_gcn1, b_gcn1, W_gcn2, b_gcn2, W_mlp1, b_mlp1, W_mlp2, b_mlp2):
    raise NotImplementedError("write your pallas kernel here")


# speedup vs baseline: 8.7266x; 8.7266x over previous
import functools

import jax
import jax.numpy as jnp
from jax import lax
from jax.experimental import pallas as pl
from jax.experimental.pallas import tpu as pltpu
from jax.experimental.pallas import tpu_sc as plsc

N = 10000
E = 320000
IN_DIM = 128
HID = 64

NC = 2
NS = 16
NW = NC * NS
L = 16
NP = 10240
RPT = NP // NS
EPW = E // NW
CH = 80
NCH = EPW // CH
HIDP = 128


@functools.cache
def _sc_kernels():
    mesh = plsc.VectorSubcoreMesh(core_axis_name="c", subcore_axis_name="s",
                                  num_cores=NC, num_subcores=NS)
    params = pltpu.CompilerParams(needs_layout_passes=False)
    deg = functools.partial(
        pl.kernel,
        out_type=jax.ShapeDtypeStruct((NW, NP), jnp.float32),
        mesh=mesh,
        compiler_params=params,
        scratch_types=[
            pltpu.VMEM((EPW,), jnp.int32),
            pltpu.VMEM((NP,), jnp.float32),
        ],
    )(_deg_body)
    mp = functools.partial(
        pl.kernel,
        out_type=jax.ShapeDtypeStruct((NC, NP, HIDP), jnp.float32),
        mesh=mesh,
        compiler_params=params,
        scratch_types=[
            pltpu.VMEM((CH,), jnp.int32),
            pltpu.VMEM((CH,), jnp.int32),
            pltpu.VMEM((CH, HIDP), jnp.float32),
            pltpu.VMEM((L, HIDP), jnp.float32),
            pltpu.VMEM_SHARED((NP, HIDP), jnp.float32),
            pltpu.SemaphoreType.DMA,
        ],
    )(_mp_body)
    eg = functools.partial(
        pl.kernel,
        out_type=(jax.ShapeDtypeStruct((E, HIDP), jnp.float32),
                  jax.ShapeDtypeStruct((E, HIDP), jnp.float32)),
        mesh=mesh,
        compiler_params=params,
        scratch_types=[
            pltpu.VMEM((CH,), jnp.int32),
            pltpu.VMEM((CH,), jnp.int32),
            pltpu.VMEM((CH, HIDP), jnp.float32),
            pltpu.VMEM((CH, HIDP), jnp.float32),
            pltpu.SemaphoreType.DMA,
            pltpu.SemaphoreType.DMA,
        ],
    )(_eg_body)
    return deg, mp, eg


def _deg_body(dst_hbm, out_hbm, idx_v, deg_v):
    cid = lax.axis_index("c")
    sid = lax.axis_index("s")
    wid = sid * NC + cid
    zero = jnp.zeros((L,), jnp.float32)

    def _z(i, c):
        deg_v[pl.ds(i * L, L)] = zero
        return c
    lax.fori_loop(0, NP // L, _z, 0)

    pltpu.sync_copy(dst_hbm.at[pl.ds(wid * EPW, EPW)], idx_v)
    one = jnp.ones((L,), jnp.float32)

    def _a(i, c):
        plsc.addupdate_scatter(deg_v, [idx_v[pl.ds(i * L, L)]], one)
        return c
    lax.fori_loop(0, EPW // L, _a, 0)

    pltpu.sync_copy(deg_v, out_hbm.at[wid])


def _mp_body(src_hbm, dst_hbm, y_hbm, out_hbm, sidx, didx, rows, ztile, acc, sem):
    cid = lax.axis_index("c")
    sid = lax.axis_index("s")
    wid = sid * NC + cid

    zero = jnp.zeros((L,), jnp.float32)
    for r in range(L):
        for c4 in range(HIDP // L):
            ztile[r, pl.ds(c4 * L, L)] = zero

    def _z(k, c):
        pltpu.sync_copy(ztile, acc.at[pl.ds(sid * RPT + k * L, L), :])
        return c
    lax.fori_loop(0, RPT // L, _z, 0)
    plsc.subcore_barrier()

    base = wid * EPW

    def _chunk(k, c):
        off = base + k * CH
        pltpu.sync_copy(src_hbm.at[pl.ds(off, CH)], sidx)
        pltpu.sync_copy(dst_hbm.at[pl.ds(off, CH)], didx)
        pltpu.async_copy(y_hbm.at[sidx], rows, sem).wait()
        pltpu.sync_copy(rows, acc.at[didx], add=True)
        return c
    lax.fori_loop(0, NCH, _chunk, 0)

    plsc.subcore_barrier()
    pltpu.sync_copy(acc.at[pl.ds(sid * RPT, RPT), :],
                    out_hbm.at[cid, pl.ds(sid * RPT, RPT), :])


def _eg_body(e0_hbm, e1_hbm, h_hbm, u_hbm, v_hbm, i0, i1, r0, r1, s0, s1):
    cid = lax.axis_index("c")
    sid = lax.axis_index("s")
    wid = sid * NC + cid
    base = wid * EPW

    def _chunk(k, c):
        off = base + k * CH
        pltpu.sync_copy(e0_hbm.at[pl.ds(off, CH)], i0)
        pltpu.sync_copy(e1_hbm.at[pl.ds(off, CH)], i1)
        d0 = pltpu.async_copy(h_hbm.at[i0], r0, s0)
        d1 = pltpu.async_copy(h_hbm.at[i1], r1, s1)
        d0.wait()
        d1.wait()
        pltpu.sync_copy(r0, u_hbm.at[pl.ds(off, CH), :])
        pltpu.sync_copy(r1, v_hbm.at[pl.ds(off, CH), :])
        return c
    lax.fori_loop(0, NCH, _chunk, 0)


def _dis_col(degp):
    ones = jnp.ones((NW, 1), jnp.float32)
    deg = lax.dot_general(degp, ones, (((0,), (0,)), ((), ())),
                          preferred_element_type=jnp.float32)
    return lax.rsqrt(deg[:N] + 1.0)


def _pad_cols(a):
    return jnp.concatenate(
        [a, jnp.zeros((a.shape[0], HIDP - HID), jnp.float32)], axis=1)


def _prep1_body(degp_ref, x_ref, w_ref, y_ref):
    dis = _dis_col(degp_ref[...])
    y = jnp.dot(x_ref[...], w_ref[...],
                preferred_element_type=jnp.float32) * dis
    y_ref[...] = _pad_cols(y)


_prep1 = pl.pallas_call(
    _prep1_body,
    out_shape=jax.ShapeDtypeStruct((N, HIDP), jnp.float32),
)


def _mid_body(degp_ref, accp_ref, y1_ref, b1_ref, w2_ref, y2_ref):
    dis = _dis_col(degp_ref[...])
    acc = accp_ref[0, :N, 0:HID] + accp_ref[1, :N, 0:HID]
    h1 = jnp.maximum((acc + y1_ref[:, 0:HID]) * dis + b1_ref[...], 0.0)
    y2 = jnp.dot(h1, w2_ref[...], preferred_element_type=jnp.float32) * dis
    y2_ref[...] = _pad_cols(y2)


_mid = pl.pallas_call(
    _mid_body,
    out_shape=jax.ShapeDtypeStruct((N, HIDP), jnp.float32),
)


def _fin_body(degp_ref, accp_ref, y2_ref, b2_ref, h2_ref):
    dis = _dis_col(degp_ref[...])
    acc = accp_ref[0, :N, 0:HID] + accp_ref[1, :N, 0:HID]
    h2 = jnp.maximum((acc + y2_ref[:, 0:HID]) * dis + b2_ref[...], 0.0)
    h2_ref[...] = _pad_cols(h2)


_fin = pl.pallas_call(
    _fin_body,
    out_shape=jax.ShapeDtypeStruct((N, HIDP), jnp.float32),
)


BE = 8000


def _mlp_body(u_ref, v_ref, w1_ref, b1_ref, w2_ref, b2_ref, o_ref):
    u = u_ref[:, 0:HID]
    v = v_ref[:, 0:HID]
    w1 = w1_ref[...]
    t = jnp.dot(u, w1[0:HID], preferred_element_type=jnp.float32)
    t = t + jnp.dot(v, w1[HID:2 * HID], preferred_element_type=jnp.float32)
    t = t + jnp.dot(jnp.abs(u - v), w1[2 * HID:3 * HID],
                    preferred_element_type=jnp.float32)
    t = t + jnp.dot(u * v, w1[3 * HID:4 * HID],
                    preferred_element_type=jnp.float32)
    t = jnp.maximum(t + b1_ref[...], 0.0)
    o_ref[...] = jnp.dot(t, w2_ref[...],
                         preferred_element_type=jnp.float32) + b2_ref[...]


_mlp = pl.pallas_call(
    _mlp_body,
    grid=(E // BE,),
    in_specs=[
        pl.BlockSpec((BE, HIDP), lambda i: (i, 0)),
        pl.BlockSpec((BE, HIDP), lambda i: (i, 0)),
        pl.BlockSpec((4 * HID, HID), lambda i: (0, 0)),
        pl.BlockSpec((HID,), lambda i: (0,)),
        pl.BlockSpec((HID, 1), lambda i: (0, 0)),
        pl.BlockSpec((1,), lambda i: (0,)),
    ],
    out_specs=pl.BlockSpec((BE, 1), lambda i: (i, 0)),
    out_shape=jax.ShapeDtypeStruct((E, 1), jnp.float32),
)


def kernel(x, graph_edge_index, edge_pairs, W_gcn1, b_gcn1, W_gcn2, b_gcn2,
           W_mlp1, b_mlp1, W_mlp2, b_mlp2):
    _deg_sc, _mp_sc, _eg_sc = _sc_kernels()
    src = graph_edge_index[0]
    dst = graph_edge_index[1]
    degp = _deg_sc(dst)
    y1 = _prep1(degp, x, W_gcn1)
    accp1 = _mp_sc(src, dst, y1)
    y2 = _mid(degp, accp1, y1, b_gcn1, W_gcn2)
    accp2 = _mp_sc(src, dst, y2)
    h2 = _fin(degp, accp2, y2, b_gcn2)
    u, v = _eg_sc(edge_pairs[0], edge_pairs[1], h2)
    out = _mlp(u, v, W_mlp1, b_mlp1, W_mlp2, b_mlp2)
    return out.reshape(-1)


# speedup vs baseline: 13.7493x; 1.5756x over previous
import functools

import jax
import jax.numpy as jnp
from jax import lax
from jax.experimental import pallas as pl
from jax.experimental.pallas import tpu as pltpu
from jax.experimental.pallas import tpu_sc as plsc

N = 10000
E = 320000
IN_DIM = 128
HID = 64

NC = 2
NS = 16
NW = NC * NS
L = 16
NP = 10240
RPT = NP // NS
EPW = E // NW
CH = 80
NCH = EPW // CH
HIDP = 128


@functools.cache
def _sc_kernels():
    mesh = plsc.VectorSubcoreMesh(core_axis_name="c", subcore_axis_name="s",
                                  num_cores=NC, num_subcores=NS)
    params = pltpu.CompilerParams(needs_layout_passes=False)
    deg = functools.partial(
        pl.kernel,
        out_type=jax.ShapeDtypeStruct((NW, NP), jnp.float32),
        mesh=mesh,
        compiler_params=params,
        scratch_types=[
            pltpu.VMEM((EPW,), jnp.int32),
            pltpu.VMEM((NP,), jnp.float32),
        ],
    )(_deg_body)
    mp = functools.partial(
        pl.kernel,
        out_type=jax.ShapeDtypeStruct((NC, NP, HIDP), jnp.float32),
        mesh=mesh,
        compiler_params=params,
        scratch_types=[
            pltpu.VMEM((EPW,), jnp.int32),
            pltpu.VMEM((NCH, CH), jnp.int32),
            pltpu.VMEM((CH, HIDP), jnp.float32),
            pltpu.VMEM((CH, HIDP), jnp.float32),
            pltpu.VMEM_SHARED((NP, HIDP), jnp.float32),
            pltpu.SemaphoreType.DMA,
            pltpu.SemaphoreType.DMA,
        ],
    )(_mp_body)
    eg = functools.partial(
        pl.kernel,
        out_type=(jax.ShapeDtypeStruct((E, HIDP), jnp.float32),
                  jax.ShapeDtypeStruct((E, HIDP), jnp.float32)),
        mesh=mesh,
        compiler_params=params,
        scratch_types=[
            pltpu.VMEM((NCH, CH), jnp.int32),
            pltpu.VMEM((NCH, CH), jnp.int32),
            pltpu.VMEM((CH, HIDP), jnp.float32),
            pltpu.VMEM((CH, HIDP), jnp.float32),
            pltpu.VMEM((CH, HIDP), jnp.float32),
            pltpu.VMEM((CH, HIDP), jnp.float32),
            pltpu.SemaphoreType.DMA,
            pltpu.SemaphoreType.DMA,
            pltpu.SemaphoreType.DMA,
            pltpu.SemaphoreType.DMA,
        ],
    )(_eg_body)
    return deg, mp, eg


def _deg_body(dst_hbm, out_hbm, idx_v, deg_v):
    cid = lax.axis_index("c")
    sid = lax.axis_index("s")
    wid = sid * NC + cid
    zero = jnp.zeros((L,), jnp.float32)

    def _z(i, c):
        deg_v[pl.ds(i * L, L)] = zero
        return c
    lax.fori_loop(0, NP // L, _z, 0)

    pltpu.sync_copy(dst_hbm.at[pl.ds(wid * EPW, EPW)], idx_v)
    one = jnp.ones((L,), jnp.float32)

    def _a(i, c):
        plsc.addupdate_scatter(deg_v, [idx_v[pl.ds(i * L, L)]], one)
        return c
    lax.fori_loop(0, EPW // L, _a, 0)

    pltpu.sync_copy(deg_v, out_hbm.at[wid])


def _mp_body(src_hbm, dst_hbm, y_hbm, out_hbm, sidx1, didx2, rows0, rows1,
             acc, gs0, gs1):
    cid = lax.axis_index("c")
    sid = lax.axis_index("s")
    wid = sid * NC + cid

    zero = jnp.zeros((L,), jnp.float32)

    def _zr(r, c):
        for c4 in range(HIDP // L):
            rows0[r, pl.ds(c4 * L, L)] = zero
        return c
    lax.fori_loop(0, CH, _zr, 0)

    def _z(j, c):
        pltpu.sync_copy(rows0, acc.at[pl.ds(sid * RPT + j * CH, CH), :])
        return c
    lax.fori_loop(0, RPT // CH, _z, 0)

    pltpu.sync_copy(src_hbm.at[pl.ds(wid * EPW, EPW)], sidx1)
    pltpu.sync_copy(dst_hbm.at[wid], didx2)
    plsc.subcore_barrier()

    pltpu.async_copy(y_hbm.at[sidx1.at[pl.ds(0, CH)]], rows0, gs0)

    def _pair(p, c):
        k0 = 2 * p
        k1 = k0 + 1
        pltpu.make_async_copy(
            y_hbm.at[sidx1.at[pl.ds(k0 * CH, CH)]], rows0, gs0).wait()
        pltpu.async_copy(y_hbm.at[sidx1.at[pl.ds(k1 * CH, CH)]], rows1, gs1)
        pltpu.sync_copy(rows0, acc.at[didx2.at[k0]], add=True)
        pltpu.make_async_copy(
            y_hbm.at[sidx1.at[pl.ds(k1 * CH, CH)]], rows1, gs1).wait()

        @pl.when(k0 + 2 < NCH)
        def _():
            pltpu.async_copy(
                y_hbm.at[sidx1.at[pl.ds((k0 + 2) * CH, CH)]], rows0, gs0)
        pltpu.sync_copy(rows1, acc.at[didx2.at[k1]], add=True)
        return c
    lax.fori_loop(0, NCH // 2, _pair, 0)

    pltpu.make_async_copy(
        y_hbm.at[sidx1.at[pl.ds((NCH - 1) * CH, CH)]], rows0, gs0).wait()
    pltpu.sync_copy(rows0, acc.at[didx2.at[NCH - 1]], add=True)

    plsc.subcore_barrier()
    pltpu.sync_copy(acc.at[pl.ds(sid * RPT, RPT), :],
                    out_hbm.at[cid, pl.ds(sid * RPT, RPT), :])


def _eg_body(e0_hbm, e1_hbm, h_hbm, u_hbm, v_hbm, i02, i12,
             ru0, ru1, rv0, rv1, su0, su1, sv0, sv1):
    cid = lax.axis_index("c")
    sid = lax.axis_index("s")
    wid = sid * NC + cid
    base = wid * EPW
    pltpu.sync_copy(e0_hbm.at[wid], i02)
    pltpu.sync_copy(e1_hbm.at[wid], i12)

    pltpu.async_copy(h_hbm.at[i02.at[0]], ru0, su0)
    pltpu.async_copy(h_hbm.at[i12.at[0]], rv0, sv0)

    def _pair(p, c):
        k0 = 2 * p
        k1 = k0 + 1
        off0 = base + k0 * CH
        off1 = base + k1 * CH
        pltpu.make_async_copy(h_hbm.at[i02.at[k0]], ru0, su0).wait()
        pltpu.async_copy(h_hbm.at[i02.at[k1]], ru1, su1)
        pltpu.sync_copy(ru0, u_hbm.at[pl.ds(off0, CH), :])
        pltpu.make_async_copy(h_hbm.at[i12.at[k0]], rv0, sv0).wait()
        pltpu.async_copy(h_hbm.at[i12.at[k1]], rv1, sv1)
        pltpu.sync_copy(rv0, v_hbm.at[pl.ds(off0, CH), :])

        pltpu.make_async_copy(h_hbm.at[i02.at[k1]], ru1, su1).wait()

        @pl.when(k0 + 2 < NCH)
        def _():
            pltpu.async_copy(h_hbm.at[i02.at[k0 + 2]], ru0, su0)
        pltpu.sync_copy(ru1, u_hbm.at[pl.ds(off1, CH), :])
        pltpu.make_async_copy(h_hbm.at[i12.at[k1]], rv1, sv1).wait()

        @pl.when(k0 + 2 < NCH)
        def _():
            pltpu.async_copy(h_hbm.at[i12.at[k0 + 2]], rv0, sv0)
        pltpu.sync_copy(rv1, v_hbm.at[pl.ds(off1, CH), :])
        return c
    lax.fori_loop(0, NCH // 2, _pair, 0)

    offl = base + (NCH - 1) * CH
    pltpu.make_async_copy(h_hbm.at[i02.at[NCH - 1]], ru0, su0).wait()
    pltpu.sync_copy(ru0, u_hbm.at[pl.ds(offl, CH), :])
    pltpu.make_async_copy(h_hbm.at[i12.at[NCH - 1]], rv0, sv0).wait()
    pltpu.sync_copy(rv0, v_hbm.at[pl.ds(offl, CH), :])


def _dis_col(degp):
    ones = jnp.ones((NW, 1), jnp.float32)
    deg = lax.dot_general(degp, ones, (((0,), (0,)), ((), ())),
                          preferred_element_type=jnp.float32)
    return lax.rsqrt(deg[:N] + 1.0)


def _pad_cols(a):
    return jnp.concatenate(
        [a, jnp.zeros((a.shape[0], HIDP - HID), jnp.float32)], axis=1)


def _prep1_body(degp_ref, x_ref, w_ref, y_ref):
    dis = _dis_col(degp_ref[...])
    y = jnp.dot(x_ref[...], w_ref[...],
                preferred_element_type=jnp.float32) * dis
    y_ref[...] = _pad_cols(y)


_prep1 = pl.pallas_call(
    _prep1_body,
    out_shape=jax.ShapeDtypeStruct((N, HIDP), jnp.float32),
)


def _mid_body(degp_ref, accp_ref, y1_ref, b1_ref, w2_ref, y2_ref):
    dis = _dis_col(degp_ref[...])
    acc = accp_ref[0, :N, 0:HID] + accp_ref[1, :N, 0:HID]
    h1 = jnp.maximum((acc + y1_ref[:, 0:HID]) * dis + b1_ref[...], 0.0)
    y2 = jnp.dot(h1, w2_ref[...], preferred_element_type=jnp.float32) * dis
    y2_ref[...] = _pad_cols(y2)


_mid = pl.pallas_call(
    _mid_body,
    out_shape=jax.ShapeDtypeStruct((N, HIDP), jnp.float32),
)


def _fin_body(degp_ref, accp_ref, y2_ref, b2_ref, h2_ref):
    dis = _dis_col(degp_ref[...])
    acc = accp_ref[0, :N, 0:HID] + accp_ref[1, :N, 0:HID]
    h2 = jnp.maximum((acc + y2_ref[:, 0:HID]) * dis + b2_ref[...], 0.0)
    h2_ref[...] = _pad_cols(h2)


_fin = pl.pallas_call(
    _fin_body,
    out_shape=jax.ShapeDtypeStruct((N, HIDP), jnp.float32),
)


BE = 8000


def _mlp_body(u_ref, v_ref, w1_ref, b1_ref, w2_ref, b2_ref, o_ref):
    u = u_ref[:, 0:HID]
    v = v_ref[:, 0:HID]
    w1 = w1_ref[...]
    t = jnp.dot(u, w1[0:HID], preferred_element_type=jnp.float32)
    t = t + jnp.dot(v, w1[HID:2 * HID], preferred_element_type=jnp.float32)
    t = t + jnp.dot(jnp.abs(u - v), w1[2 * HID:3 * HID],
                    preferred_element_type=jnp.float32)
    t = t + jnp.dot(u * v, w1[3 * HID:4 * HID],
                    preferred_element_type=jnp.float32)
    t = jnp.maximum(t + b1_ref[...], 0.0)
    o_ref[...] = jnp.dot(t, w2_ref[...],
                         preferred_element_type=jnp.float32) + b2_ref[...]


_mlp = pl.pallas_call(
    _mlp_body,
    grid=(E // BE,),
    in_specs=[
        pl.BlockSpec((BE, HIDP), lambda i: (i, 0)),
        pl.BlockSpec((BE, HIDP), lambda i: (i, 0)),
        pl.BlockSpec((4 * HID, HID), lambda i: (0, 0)),
        pl.BlockSpec((HID,), lambda i: (0,)),
        pl.BlockSpec((HID, 1), lambda i: (0, 0)),
        pl.BlockSpec((1,), lambda i: (0,)),
    ],
    out_specs=pl.BlockSpec((BE, 1), lambda i: (i, 0)),
    out_shape=jax.ShapeDtypeStruct((E, 1), jnp.float32),
)


def kernel(x, graph_edge_index, edge_pairs, W_gcn1, b_gcn1, W_gcn2, b_gcn2,
           W_mlp1, b_mlp1, W_mlp2, b_mlp2):
    _deg_sc, _mp_sc, _eg_sc = _sc_kernels()
    src = graph_edge_index[0]
    dst = graph_edge_index[1]
    dst2 = dst.reshape(NW, NCH, CH)
    e02 = edge_pairs[0].reshape(NW, NCH, CH)
    e12 = edge_pairs[1].reshape(NW, NCH, CH)
    degp = _deg_sc(dst)
    y1 = _prep1(degp, x, W_gcn1)
    accp1 = _mp_sc(src, dst2, y1)
    y2 = _mid(degp, accp1, y1, b_gcn1, W_gcn2)
    accp2 = _mp_sc(src, dst2, y2)
    h2 = _fin(degp, accp2, y2, b_gcn2)
    u, v = _eg_sc(e02, e12, h2)
    out = _mlp(u, v, W_mlp1, b_mlp1, W_mlp2, b_mlp2)
    return out.reshape(-1)


# speedup vs baseline: 14.7493x; 1.0727x over previous
import functools

import jax
import jax.numpy as jnp
from jax import lax
from jax.experimental import pallas as pl
from jax.experimental.pallas import tpu as pltpu
from jax.experimental.pallas import tpu_sc as plsc

N = 10000
E = 320000
IN_DIM = 128
HID = 64

NC = 2
NS = 16
NW = NC * NS
L = 16
NP = 10240
RPT = NP // NS
EPW = E // NW
CH = 80
NCH = EPW // CH
HIDP = 128


@functools.cache
def _sc_kernels():
    mesh = plsc.VectorSubcoreMesh(core_axis_name="c", subcore_axis_name="s",
                                  num_cores=NC, num_subcores=NS)
    params = pltpu.CompilerParams(needs_layout_passes=False)
    deg = functools.partial(
        pl.kernel,
        out_type=jax.ShapeDtypeStruct((NW, NP), jnp.float32),
        mesh=mesh,
        compiler_params=params,
        scratch_types=[
            pltpu.VMEM((EPW,), jnp.int32),
            pltpu.VMEM((NP,), jnp.float32),
        ],
    )(_deg_body)
    mp = functools.partial(
        pl.kernel,
        out_type=jax.ShapeDtypeStruct((NC, NP, HIDP), jnp.float32),
        mesh=mesh,
        compiler_params=params,
        scratch_types=[
            pltpu.VMEM((EPW,), jnp.int32),
            pltpu.VMEM((NCH, CH), jnp.int32),
            pltpu.VMEM((CH, HIDP), jnp.float32),
            pltpu.VMEM((CH, HIDP), jnp.float32),
            pltpu.VMEM_SHARED((NP, HIDP), jnp.float32),
            pltpu.SemaphoreType.DMA,
            pltpu.SemaphoreType.DMA,
        ],
    )(_mp_body)
    eg = functools.partial(
        pl.kernel,
        out_type=jax.ShapeDtypeStruct((E, HIDP), jnp.float32),
        mesh=mesh,
        compiler_params=params,
        scratch_types=[
            pltpu.VMEM((EPW,), jnp.int32),
            pltpu.VMEM((EPW,), jnp.int32),
            pltpu.VMEM((CH, HIDP), jnp.float32),
            pltpu.VMEM((CH, HIDP), jnp.float32),
            pltpu.VMEM((CH, HIDP), jnp.float32),
            pltpu.VMEM((CH, HIDP), jnp.float32),
            pltpu.VMEM((CH, HIDP), jnp.float32),
            pltpu.VMEM((CH, HIDP), jnp.float32),
            pltpu.SemaphoreType.DMA,
            pltpu.SemaphoreType.DMA,
            pltpu.SemaphoreType.DMA,
            pltpu.SemaphoreType.DMA,
        ],
    )(_eg_body)
    return deg, mp, eg


def _deg_body(dst_hbm, out_hbm, idx_v, deg_v):
    cid = lax.axis_index("c")
    sid = lax.axis_index("s")
    wid = sid * NC + cid
    zero = jnp.zeros((L,), jnp.float32)

    def _z(i, c):
        deg_v[pl.ds(i * L, L)] = zero
        return c
    lax.fori_loop(0, NP // L, _z, 0)

    pltpu.sync_copy(dst_hbm.at[pl.ds(wid * EPW, EPW)], idx_v)
    one = jnp.ones((L,), jnp.float32)

    def _a(i, c):
        plsc.addupdate_scatter(deg_v, [idx_v[pl.ds(i * L, L)]], one)
        return c
    lax.fori_loop(0, EPW // L, _a, 0)

    pltpu.sync_copy(deg_v, out_hbm.at[wid])


def _mp_body(src_hbm, dst_hbm, y_hbm, out_hbm, sidx1, didx2, rows0, rows1,
             acc, gs0, gs1):
    cid = lax.axis_index("c")
    sid = lax.axis_index("s")
    wid = sid * NC + cid

    zero = jnp.zeros((L,), jnp.float32)

    def _zr(r, c):
        for c4 in range(HIDP // L):
            rows0[r, pl.ds(c4 * L, L)] = zero
        return c
    lax.fori_loop(0, CH, _zr, 0)

    def _z(j, c):
        pltpu.sync_copy(rows0, acc.at[pl.ds(sid * RPT + j * CH, CH), :])
        return c
    lax.fori_loop(0, RPT // CH, _z, 0)

    pltpu.sync_copy(src_hbm.at[pl.ds(wid * EPW, EPW)], sidx1)
    pltpu.sync_copy(dst_hbm.at[wid], didx2)
    plsc.subcore_barrier()

    pltpu.async_copy(y_hbm.at[sidx1.at[pl.ds(0, CH)]], rows0, gs0)

    def _pair(p, c):
        k0 = 2 * p
        k1 = k0 + 1
        pltpu.make_async_copy(
            y_hbm.at[sidx1.at[pl.ds(k0 * CH, CH)]], rows0, gs0).wait()
        pltpu.async_copy(y_hbm.at[sidx1.at[pl.ds(k1 * CH, CH)]], rows1, gs1)
        pltpu.sync_copy(rows0, acc.at[didx2.at[k0]], add=True)
        pltpu.make_async_copy(
            y_hbm.at[sidx1.at[pl.ds(k1 * CH, CH)]], rows1, gs1).wait()

        @pl.when(k0 + 2 < NCH)
        def _():
            pltpu.async_copy(
                y_hbm.at[sidx1.at[pl.ds((k0 + 2) * CH, CH)]], rows0, gs0)
        pltpu.sync_copy(rows1, acc.at[didx2.at[k1]], add=True)
        return c
    lax.fori_loop(0, NCH // 2, _pair, 0)

    pltpu.make_async_copy(
        y_hbm.at[sidx1.at[pl.ds((NCH - 1) * CH, CH)]], rows0, gs0).wait()
    pltpu.sync_copy(rows0, acc.at[didx2.at[NCH - 1]], add=True)

    plsc.subcore_barrier()
    pltpu.sync_copy(acc.at[pl.ds(sid * RPT, RPT), :],
                    out_hbm.at[cid, pl.ds(sid * RPT, RPT), :])


def _eg_body(e0_hbm, e1_hbm, h_hbm, uv_hbm, i0, i1,
             ru0, ru1, rv0, rv1, rp0, rp1, su0, su1, sv0, sv1):
    cid = lax.axis_index("c")
    sid = lax.axis_index("s")
    wid = sid * NC + cid
    base = wid * EPW
    pltpu.sync_copy(e0_hbm.at[pl.ds(base, EPW)], i0)
    pltpu.sync_copy(e1_hbm.at[pl.ds(base, EPW)], i1)

    def _gath(k, ru, su, rv, sv):
        pltpu.async_copy(h_hbm.at[i0.at[pl.ds(k * CH, CH)]], ru, su)
        pltpu.async_copy(h_hbm.at[i1.at[pl.ds(k * CH, CH)]], rv, sv)

    def _wait(k, ru, su, rv, sv):
        pltpu.make_async_copy(h_hbm.at[i0.at[pl.ds(k * CH, CH)]], ru, su).wait()
        pltpu.make_async_copy(h_hbm.at[i1.at[pl.ds(k * CH, CH)]], rv, sv).wait()

    def _pack(ru, rv, rp):
        def _row(r, c):
            for c4 in range(HID // L):
                rp[r, pl.ds(c4 * L, L)] = ru[r, pl.ds(c4 * L, L)]
                rp[r, pl.ds(HID + c4 * L, L)] = rv[r, pl.ds(c4 * L, L)]
            return c
        lax.fori_loop(0, CH, _row, 0)

    _gath(0, ru0, su0, rv0, sv0)

    def _pair(p, c):
        k0 = 2 * p
        k1 = k0 + 1
        _wait(k0, ru0, su0, rv0, sv0)
        _gath(k1, ru1, su1, rv1, sv1)
        _pack(ru0, rv0, rp0)
        pltpu.sync_copy(rp0, uv_hbm.at[pl.ds(base + k0 * CH, CH), :])
        _wait(k1, ru1, su1, rv1, sv1)

        @pl.when(k0 + 2 < NCH)
        def _():
            _gath(k0 + 2, ru0, su0, rv0, sv0)
        _pack(ru1, rv1, rp1)
        pltpu.sync_copy(rp1, uv_hbm.at[pl.ds(base + k1 * CH, CH), :])
        return c
    lax.fori_loop(0, NCH // 2, _pair, 0)

    _wait(NCH - 1, ru0, su0, rv0, sv0)
    _pack(ru0, rv0, rp0)
    pltpu.sync_copy(rp0, uv_hbm.at[pl.ds(base + (NCH - 1) * CH, CH), :])


def _dis_col(degp):
    ones = jnp.ones((NW, 1), jnp.float32)
    deg = lax.dot_general(degp, ones, (((0,), (0,)), ((), ())),
                          preferred_element_type=jnp.float32)
    return lax.rsqrt(deg[:N] + 1.0)


def _pad_cols(a):
    return jnp.concatenate(
        [a, jnp.zeros((a.shape[0], HIDP - HID), jnp.float32)], axis=1)


def _prep1_body(degp_ref, x_ref, w_ref, y_ref):
    dis = _dis_col(degp_ref[...])
    y = jnp.dot(x_ref[...], w_ref[...],
                preferred_element_type=jnp.float32) * dis
    y_ref[...] = _pad_cols(y)


_prep1 = pl.pallas_call(
    _prep1_body,
    out_shape=jax.ShapeDtypeStruct((N, HIDP), jnp.float32),
)


def _mid_body(degp_ref, accp_ref, y1_ref, b1_ref, w2_ref, y2_ref):
    dis = _dis_col(degp_ref[...])
    acc = accp_ref[0, :N, 0:HID] + accp_ref[1, :N, 0:HID]
    h1 = jnp.maximum((acc + y1_ref[:, 0:HID]) * dis + b1_ref[...], 0.0)
    y2 = jnp.dot(h1, w2_ref[...], preferred_element_type=jnp.float32) * dis
    y2_ref[...] = _pad_cols(y2)


_mid = pl.pallas_call(
    _mid_body,
    out_shape=jax.ShapeDtypeStruct((N, HIDP), jnp.float32),
)


def _fin_body(degp_ref, accp_ref, y2_ref, b2_ref, h2_ref):
    dis = _dis_col(degp_ref[...])
    acc = accp_ref[0, :N, 0:HID] + accp_ref[1, :N, 0:HID]
    h2 = jnp.maximum((acc + y2_ref[:, 0:HID]) * dis + b2_ref[...], 0.0)
    h2_ref[...] = _pad_cols(h2)


_fin = pl.pallas_call(
    _fin_body,
    out_shape=jax.ShapeDtypeStruct((N, HIDP), jnp.float32),
)


BE = 8000


def _mlp_body(uv_ref, w1_ref, b1_ref, w2_ref, b2_ref, o_ref):
    u = uv_ref[:, 0:HID]
    v = uv_ref[:, HID:2 * HID]
    w1 = w1_ref[...]
    t = jnp.dot(u, w1[0:HID], preferred_element_type=jnp.float32)
    t = t + jnp.dot(v, w1[HID:2 * HID], preferred_element_type=jnp.float32)
    t = t + jnp.dot(jnp.abs(u - v), w1[2 * HID:3 * HID],
                    preferred_element_type=jnp.float32)
    t = t + jnp.dot(u * v, w1[3 * HID:4 * HID],
                    preferred_element_type=jnp.float32)
    t = jnp.maximum(t + b1_ref[...], 0.0)
    o_ref[...] = jnp.dot(t, w2_ref[...],
                         preferred_element_type=jnp.float32) + b2_ref[...]


_mlp = pl.pallas_call(
    _mlp_body,
    grid=(E // BE,),
    in_specs=[
        pl.BlockSpec((BE, HIDP), lambda i: (i, 0)),
        pl.BlockSpec((4 * HID, HID), lambda i: (0, 0)),
        pl.BlockSpec((HID,), lambda i: (0,)),
        pl.BlockSpec((HID, 1), lambda i: (0, 0)),
        pl.BlockSpec((1,), lambda i: (0,)),
    ],
    out_specs=pl.BlockSpec((BE, 1), lambda i: (i, 0)),
    out_shape=jax.ShapeDtypeStruct((E, 1), jnp.float32),
)


def kernel(x, graph_edge_index, edge_pairs, W_gcn1, b_gcn1, W_gcn2, b_gcn2,
           W_mlp1, b_mlp1, W_mlp2, b_mlp2):
    _deg_sc, _mp_sc, _eg_sc = _sc_kernels()
    src = graph_edge_index[0]
    dst = graph_edge_index[1]
    dst2 = dst.reshape(NW, NCH, CH)

    degp = _deg_sc(dst)
    y1 = _prep1(degp, x, W_gcn1)
    accp1 = _mp_sc(src, dst2, y1)
    y2 = _mid(degp, accp1, y1, b_gcn1, W_gcn2)
    accp2 = _mp_sc(src, dst2, y2)
    h2 = _fin(degp, accp2, y2, b_gcn2)
    uv = _eg_sc(edge_pairs[0], edge_pairs[1], h2)
    out = _mlp(uv, W_mlp1, b_mlp1, W_mlp2, b_mlp2)
    return out.reshape(-1)
